# baseline (speedup 1.0000x reference)
import functools

import jax
import jax.numpy as jnp
from jax import lax
from jax.experimental import pallas as pl
from jax.experimental.pallas import tpu as pltpu
from jax.experimental.pallas import tpu_sc as plsc

B, L, D = 4, 8192, 768
NC, NS, LANES = 2, 16, 16
NW = NC * NS
ROWS_W = L // NW
T = 32
STEPS = ROWS_W // T
CHUNKS = T * D // LANES
UNROLL = 8


def _sc_body(x_hbm, t_hbm, o_hbm, t_v, x_v):
    wid = lax.axis_index("s") * NC + lax.axis_index("c")
    base = wid * ROWS_W * D

    def step(s, carry):
        t_off = base + s * (T * D)
        pltpu.sync_copy(t_hbm.at[pl.ds(t_off, T * D)], t_v)

        def batch(b, carry):
            x_off = b * (L * D) + t_off
            pltpu.sync_copy(x_hbm.at[pl.ds(x_off, T * D)], x_v)

            def chunk(i, carry):
                for u in range(UNROLL):
                    off = (i * UNROLL + u) * LANES
                    plsc.addupdate(x_v.at[pl.ds(off, LANES)],
                                   t_v[pl.ds(off, LANES)])
                return carry

            lax.fori_loop(0, CHUNKS // UNROLL, chunk, 0)
            pltpu.sync_copy(x_v, o_hbm.at[pl.ds(x_off, T * D)])
            return carry

        lax.fori_loop(0, B, batch, 0)
        return carry

    lax.fori_loop(0, STEPS, step, 0)


@functools.partial(
    pl.kernel,
    out_type=jax.ShapeDtypeStruct((B * L * D,), jnp.float32),
    mesh=plsc.VectorSubcoreMesh(core_axis_name="c", subcore_axis_name="s"),
    scratch_types=[
        pltpu.VMEM((T * D,), jnp.float32),
        pltpu.VMEM((T * D,), jnp.float32),
    ],
)
def _sc_add(x_hbm, t_hbm, o_hbm, t_v, x_v):
    _sc_body(x_hbm, t_hbm, o_hbm, t_v, x_v)


def kernel(x, table):
    b, l, d = x.shape
    out = _sc_add(x.reshape(-1), table.reshape(-1))
    return out.reshape(b, l, d)


# speedup vs baseline: 1.1964x; 1.1964x over previous
import functools

import jax
import jax.numpy as jnp
from jax import lax
from jax.experimental import pallas as pl
from jax.experimental.pallas import tpu as pltpu
from jax.experimental.pallas import tpu_sc as plsc

B, L, D = 4, 8192, 768
NC, NS, LANES = 2, 16, 16
NW = NC * NS
ROWS_W = L // NW
T = 32
TD = T * D
STEPS = ROWS_W // T
UNITS = [(s, b) for s in range(STEPS) for b in range(B)]


def _sc_body(x_hbm, t_hbm, o_hbm, t0, t1, x0, x1,
             s_t0, s_t1, s_xi0, s_xi1, s_xo0, s_xo1):
    t_bufs, x_bufs = (t0, t1), (x0, x1)
    s_t, s_xi, s_xo = (s_t0, s_t1), (s_xi0, s_xi1), (s_xo0, s_xo1)

    wid = lax.axis_index("s") * NC + lax.axis_index("c")
    base = wid * ROWS_W * D
    t_off = lambda s: base + s * TD
    x_off = lambda s, b: b * (L * D) + t_off(s)

    def add_tile(x_v, t_v):
        @plsc.parallel_loop(0, TD, step=LANES, unroll=8)
        def _(off):
            plsc.addupdate(x_v.at[pl.ds(off, LANES)], t_v[pl.ds(off, LANES)])

    tin = [None, None]
    xin = [None, None]
    xout = [None, None]
    tin[0] = pltpu.async_copy(t_hbm.at[pl.ds(t_off(0), TD)], t_bufs[0], s_t[0])
    xin[0] = pltpu.async_copy(x_hbm.at[pl.ds(x_off(0, 0), TD)], x_bufs[0],
                              s_xi[0])

    for u, (s, b) in enumerate(UNITS):
        cur, nxt = u % 2, (u + 1) % 2
        if u + 1 < len(UNITS):
            s2, b2 = UNITS[u + 1]
            if xout[nxt] is not None:
                xout[nxt].wait()
            xin[nxt] = pltpu.async_copy(
                x_hbm.at[pl.ds(x_off(s2, b2), TD)], x_bufs[nxt], s_xi[nxt])
            if b2 == 0:
                tin[s2 % 2] = pltpu.async_copy(
                    t_hbm.at[pl.ds(t_off(s2), TD)], t_bufs[s2 % 2],
                    s_t[s2 % 2])
        if b == 0:
            tin[s % 2].wait()
        xin[cur].wait()
        add_tile(x_bufs[cur], t_bufs[s % 2])
        xout[cur] = pltpu.async_copy(
            x_bufs[cur], o_hbm.at[pl.ds(x_off(s, b), TD)], s_xo[cur])

    xout[0].wait()
    xout[1].wait()


@functools.partial(
    pl.kernel,
    out_type=jax.ShapeDtypeStruct((B * L * D,), jnp.float32),
    mesh=plsc.VectorSubcoreMesh(core_axis_name="c", subcore_axis_name="s"),
    scratch_types=[
        pltpu.VMEM((TD,), jnp.float32),
        pltpu.VMEM((TD,), jnp.float32),
        pltpu.VMEM((TD,), jnp.float32),
        pltpu.VMEM((TD,), jnp.float32),
        pltpu.SemaphoreType.DMA,
        pltpu.SemaphoreType.DMA,
        pltpu.SemaphoreType.DMA,
        pltpu.SemaphoreType.DMA,
        pltpu.SemaphoreType.DMA,
        pltpu.SemaphoreType.DMA,
    ],
)
def _sc_add(*refs):
    _sc_body(*refs)


def kernel(x, table):
    b, l, d = x.shape
    out = _sc_add(x.reshape(-1), table.reshape(-1))
    return out.reshape(b, l, d)


# speedup vs baseline: 3.2316x; 2.7010x over previous
import functools

import jax
import jax.numpy as jnp
from jax import lax
from jax.experimental import pallas as pl
from jax.experimental.pallas import tpu as pltpu
from jax.experimental.pallas import tpu_sc as plsc

B, L, D = 4, 8192, 768
NC, NS, LANES = 2, 16, 16
NW = NC * NS
ROWS_W = L // NW
T = 32
STEPS = ROWS_W // T
UNITS = [(s, b) for s in range(STEPS) for b in range(B)]
CCH = D // LANES


def _sc_body(x_hbm, t_hbm, o_hbm, t0, t1, x0, x1,
             s_t0, s_t1, s_xi0, s_xi1, s_xo0, s_xo1):
    t_bufs, x_bufs = (t0, t1), (x0, x1)
    s_t, s_xi, s_xo = (s_t0, s_t1), (s_xi0, s_xi1), (s_xo0, s_xo1)

    wid = lax.axis_index("s") * NC + lax.axis_index("c")
    row_at = lambda s: wid * ROWS_W + s * T

    def add_tile(x_v, t_v):
        @plsc.parallel_loop(0, T)
        def _(r):
            @plsc.parallel_loop(0, D, step=LANES, unroll=8)
            def _(c):
                plsc.addupdate(x_v.at[r, pl.ds(c, LANES)],
                               t_v[r, pl.ds(c, LANES)])

    tin = [None, None]
    xin = [None, None]
    xout = [None, None]
    tin[0] = pltpu.async_copy(t_hbm.at[pl.ds(row_at(0), T)], t_bufs[0], s_t[0])
    xin[0] = pltpu.async_copy(x_hbm.at[0, pl.ds(row_at(0), T)], x_bufs[0],
                              s_xi[0])

    for u, (s, b) in enumerate(UNITS):
        cur, nxt = u % 2, (u + 1) % 2
        if u + 1 < len(UNITS):
            s2, b2 = UNITS[u + 1]
            if xout[nxt] is not None:
                xout[nxt].wait()
            xin[nxt] = pltpu.async_copy(
                x_hbm.at[b2, pl.ds(row_at(s2), T)], x_bufs[nxt], s_xi[nxt])
            if b2 == 0:
                tin[s2 % 2] = pltpu.async_copy(
                    t_hbm.at[pl.ds(row_at(s2), T)], t_bufs[s2 % 2],
                    s_t[s2 % 2])
        if b == 0:
            tin[s % 2].wait()
        xin[cur].wait()
        add_tile(x_bufs[cur], t_bufs[s % 2])
        xout[cur] = pltpu.async_copy(
            x_bufs[cur], o_hbm.at[b, pl.ds(row_at(s), T)], s_xo[cur])

    xout[0].wait()
    xout[1].wait()


@functools.partial(
    pl.kernel,
    out_type=jax.ShapeDtypeStruct((B, L, D), jnp.float32),
    mesh=plsc.VectorSubcoreMesh(core_axis_name="c", subcore_axis_name="s"),
    scratch_types=[
        pltpu.VMEM((T, D), jnp.float32),
        pltpu.VMEM((T, D), jnp.float32),
        pltpu.VMEM((T, D), jnp.float32),
        pltpu.VMEM((T, D), jnp.float32),
        pltpu.SemaphoreType.DMA,
        pltpu.SemaphoreType.DMA,
        pltpu.SemaphoreType.DMA,
        pltpu.SemaphoreType.DMA,
        pltpu.SemaphoreType.DMA,
        pltpu.SemaphoreType.DMA,
    ],
)
def _sc_add(*refs):
    _sc_body(*refs)


def kernel(x, table):
    return _sc_add(x, table)
